# baseline (speedup 1.0000x reference)
import functools

import jax
import jax.numpy as jnp
from jax import lax
from jax.experimental import pallas as pl
from jax.experimental.pallas import tpu as pltpu
from jax.experimental.pallas import tpu_sc as plsc

EMBED_DIM = 32
BATCH, SEQ = 16384, 50
B_TOTAL = BATCH * SEQ

_info = plsc.get_sparse_core_info()
_NC, _NS = _info.num_cores, _info.num_subcores
_NW = _NC * _NS
_B_PER_W = B_TOTAL // _NW
_CHUNK = 1600
_NCHUNK = _B_PER_W // _CHUNK

_mesh = plsc.VectorSubcoreMesh(core_axis_name="c", subcore_axis_name="s")


@functools.partial(
    pl.kernel,
    mesh=_mesh,
    out_type=jax.ShapeDtypeStruct((B_TOTAL, EMBED_DIM), jnp.float32),
    scratch_types=[
        pltpu.VMEM((2, _CHUNK), jnp.int32),
        pltpu.VMEM((2, _CHUNK, EMBED_DIM), jnp.float32),
        pltpu.SemaphoreType.DMA,
        pltpu.SemaphoreType.DMA,
    ],
    compiler_params=pltpu.CompilerParams(use_tc_tiling_on_sc=False),
)
def _gather(table_hbm, idx_hbm, out_hbm, idx_v, rows_v, sem0, sem1):
    wid = lax.axis_index("s") * _NC + lax.axis_index("c")
    base = wid * _B_PER_W
    sems = (sem0, sem1)

    def load_and_fire(g):
        b = g % 2
        pltpu.sync_copy(idx_hbm.at[pl.ds(base + g * _CHUNK, _CHUNK)], idx_v.at[b])
        return pltpu.async_copy(table_hbm.at[idx_v.at[b]], rows_v.at[b], sems[b])

    copies = [None] * _NCHUNK
    copies[0] = load_and_fire(0)
    for g in range(_NCHUNK):
        if g + 1 < _NCHUNK:
            copies[g + 1] = load_and_fire(g + 1)
        copies[g].wait()
        pltpu.sync_copy(rows_v.at[g % 2],
                        out_hbm.at[pl.ds(base + g * _CHUNK, _CHUNK)])


def kernel(x, embed_weight):
    idx = x.reshape(-1).astype(jnp.int32)
    out = _gather(embed_weight, idx)
    return out.reshape(BATCH, SEQ, EMBED_DIM)


# speedup vs baseline: 1.6226x; 1.6226x over previous
import functools

import jax
import jax.numpy as jnp
from jax import lax
from jax.experimental import pallas as pl
from jax.experimental.pallas import tpu as pltpu
from jax.experimental.pallas import tpu_sc as plsc

EMBED_DIM = 32
VOCAB_ROWS = 1000000
BATCH, SEQ = 16384, 50
B_TOTAL = BATCH * SEQ

_info = plsc.get_sparse_core_info()
_NC, _NS = _info.num_cores, _info.num_subcores
_NW = _NC * _NS
_B_PER_W = B_TOTAL // _NW
_CHUNK = 1600
_NCHUNK = _B_PER_W // _CHUNK

_mesh = plsc.VectorSubcoreMesh(core_axis_name="c", subcore_axis_name="s")


@functools.partial(
    pl.kernel,
    mesh=_mesh,
    out_type=jax.ShapeDtypeStruct((B_TOTAL, EMBED_DIM), jnp.float32),
    scratch_types=[
        pltpu.VMEM((2, _CHUNK), jnp.int32),
        pltpu.VMEM((2, _CHUNK, EMBED_DIM), jnp.float32),
        pltpu.SemaphoreType.DMA,
        pltpu.SemaphoreType.DMA,
    ],
    compiler_params=pltpu.CompilerParams(use_tc_tiling_on_sc=False),
)
def _gather(table_hbm, idx_hbm, out_hbm, idx_v, rows_v, sem0, sem1):
    wid = lax.axis_index("s") * _NC + lax.axis_index("c")
    base = wid * _B_PER_W
    sems = (sem0, sem1)

    def load_and_fire(g):
        b = g % 2
        pltpu.sync_copy(idx_hbm.at[pl.ds(base + g * _CHUNK, _CHUNK)], idx_v.at[b])
        return pltpu.async_copy(table_hbm.at[idx_v.at[b]], rows_v.at[b], sems[b])

    copies = [None] * _NCHUNK
    copies[0] = load_and_fire(0)
    for g in range(_NCHUNK):
        if g + 1 < _NCHUNK:
            copies[g + 1] = load_and_fire(g + 1)
        copies[g].wait()
        pltpu.sync_copy(rows_v.at[g % 2],
                        out_hbm.at[pl.ds(base + g * _CHUNK, _CHUNK)])


def kernel(x, embed_weight):
    t2 = jax.lax.optimization_barrier(jnp.reshape(embed_weight, (250000, 128)))
    t_lin = jnp.reshape(t2, (VOCAB_ROWS, EMBED_DIM))
    idx = x.reshape(-1).astype(jnp.int32)
    out = _gather(t_lin, idx)
    o2 = jax.lax.optimization_barrier(jnp.reshape(out, (B_TOTAL // 4, 128)))
    return jnp.reshape(o2, (BATCH, SEQ, EMBED_DIM))
